# speedup vs baseline: 135.7513x; 135.7513x over previous
import functools

import jax
import jax.numpy as jnp
from jax import lax
from jax.experimental import pallas as pl
from jax.experimental.pallas import tpu as pltpu
from jax.experimental.pallas import tpu_sc as plsc

BATCH = 16384
HIST = 200
TOTAL = BATCH * HIST

_info = plsc.get_sparse_core_info()
NC = _info.num_cores
NS = _info.num_subcores
NW = NC * NS
PER_TILE = TOTAL // NW
NCHUNK = 8
CHUNK = PER_TILE // NCHUNK

_mesh = plsc.VectorSubcoreMesh(core_axis_name="c", subcore_axis_name="s")


@functools.partial(
    pl.kernel,
    mesh=_mesh,
    out_type=jax.ShapeDtypeStruct((TOTAL,), jnp.float32),
    scratch_types=[
        pltpu.VMEM((CHUNK,), jnp.int32),
        pltpu.VMEM((CHUNK,), jnp.float32),
        pltpu.SemaphoreType.DMA,
    ],
)
def _gather_kernel(ids_hbm, table_hbm, out_hbm, idx_v, vals_v, sem):
    wid = lax.axis_index("s") * NC + lax.axis_index("c")
    base = wid * PER_TILE

    def chunk_body(i, _):
        off = base + i * CHUNK
        pltpu.sync_copy(ids_hbm.at[pl.ds(off, CHUNK)], idx_v)
        pltpu.async_copy(table_hbm.at[idx_v], vals_v, sem).wait()
        pltpu.sync_copy(vals_v, out_hbm.at[pl.ds(off, CHUNK)])
        return 0

    lax.fori_loop(0, NCHUNK, chunk_body, 0)


def kernel(input_ids, mapping_table):
    flat_ids = input_ids.reshape(TOTAL)
    out = _gather_kernel(flat_ids, mapping_table)
    return out.reshape(BATCH, HIST)


# speedup vs baseline: 139.2636x; 1.0259x over previous
import functools

import jax
import jax.numpy as jnp
from jax import lax
from jax.experimental import pallas as pl
from jax.experimental.pallas import tpu as pltpu
from jax.experimental.pallas import tpu_sc as plsc

BATCH = 16384
HIST = 200
TOTAL = BATCH * HIST

_info = plsc.get_sparse_core_info()
NC = _info.num_cores
NS = _info.num_subcores
NW = NC * NS
PER_TILE = TOTAL // NW
NCHUNK = 8
CHUNK = PER_TILE // NCHUNK

_mesh = plsc.VectorSubcoreMesh(core_axis_name="c", subcore_axis_name="s")


@functools.partial(
    pl.kernel,
    mesh=_mesh,
    out_type=jax.ShapeDtypeStruct((TOTAL,), jnp.float32),
    scratch_types=[
        pltpu.VMEM((CHUNK,), jnp.int32),
        pltpu.VMEM((CHUNK,), jnp.int32),
        pltpu.VMEM((CHUNK,), jnp.float32),
        pltpu.VMEM((CHUNK,), jnp.float32),
        pltpu.SemaphoreType.DMA,
        pltpu.SemaphoreType.DMA,
        pltpu.SemaphoreType.DMA,
        pltpu.SemaphoreType.DMA,
        pltpu.SemaphoreType.DMA,
    ],
)
def _gather_kernel(ids_hbm, table_hbm, out_hbm, idx0, idx1, vals0, vals1,
                   isem0, isem1, gsem, ssem0, ssem1):
    wid = lax.axis_index("s") * NC + lax.axis_index("c")
    base = wid * PER_TILE
    idx = (idx0, idx1)
    vals = (vals0, vals1)
    isem = (isem0, isem1)
    ssem = (ssem0, ssem1)

    for b in range(2):
        pltpu.async_copy(
            ids_hbm.at[pl.ds(base + b * CHUNK, CHUNK)], idx[b], isem[b])

    for i in range(NCHUNK):
        b = i % 2
        pltpu.make_async_copy(
            ids_hbm.at[pl.ds(base + i * CHUNK, CHUNK)], idx[b],
            isem[b]).wait()
        if i >= 2:
            pltpu.make_async_copy(
                vals[b], out_hbm.at[pl.ds(base + (i - 2) * CHUNK, CHUNK)],
                ssem[b]).wait()
        pltpu.async_copy(table_hbm.at[idx[b]], vals[b], gsem).wait()
        pltpu.async_copy(
            vals[b], out_hbm.at[pl.ds(base + i * CHUNK, CHUNK)], ssem[b])
        if i + 2 < NCHUNK:
            pltpu.async_copy(
                ids_hbm.at[pl.ds(base + (i + 2) * CHUNK, CHUNK)], idx[b],
                isem[b])

    for i in range(NCHUNK - 2, NCHUNK):
        b = i % 2
        pltpu.make_async_copy(
            vals[b], out_hbm.at[pl.ds(base + i * CHUNK, CHUNK)],
            ssem[b]).wait()


def kernel(input_ids, mapping_table):
    flat_ids = input_ids.reshape(TOTAL)
    out = _gather_kernel(flat_ids, mapping_table)
    return out.reshape(BATCH, HIST)


# speedup vs baseline: 226.1554x; 1.6239x over previous
import functools

import jax
import jax.numpy as jnp
from jax import lax
from jax.experimental import pallas as pl
from jax.experimental.pallas import tpu as pltpu
from jax.experimental.pallas import tpu_sc as plsc

VOCAB = 1000000
BATCH = 16384
HIST = 200
TOTAL = BATCH * HIST

_info = plsc.get_sparse_core_info()
NC = _info.num_cores
NS = _info.num_subcores
NW = NC * NS
PER_TILE = TOTAL // NW
NCHUNK = 8
CHUNK = PER_TILE // NCHUNK
STAGE_HOP = 10416
STAGE_NHOP = 6
STAGE = STAGE_HOP * STAGE_NHOP
STAGE_TAIL = VOCAB - 16 * STAGE

_mesh = plsc.VectorSubcoreMesh(core_axis_name="c", subcore_axis_name="s")


@functools.partial(
    pl.kernel,
    mesh=_mesh,
    out_type=jax.ShapeDtypeStruct((TOTAL,), jnp.float32),
    scratch_types=[
        pltpu.VMEM((CHUNK,), jnp.int32),
        pltpu.VMEM((CHUNK,), jnp.int32),
        pltpu.VMEM((CHUNK,), jnp.float32),
        pltpu.VMEM((CHUNK,), jnp.float32),
        pltpu.VMEM_SHARED((VOCAB,), jnp.float32),
        pltpu.VMEM((STAGE_HOP,), jnp.float32),
        pltpu.SemaphoreType.DMA,
        pltpu.SemaphoreType.DMA,
        pltpu.SemaphoreType.DMA,
        pltpu.SemaphoreType.DMA,
        pltpu.SemaphoreType.DMA,
    ],
)
def _gather_kernel(ids_hbm, table_hbm, out_hbm, idx0, idx1, vals0, vals1,
                   table_sh, bounce, isem0, isem1, gsem, ssem0, ssem1):
    sid = lax.axis_index("s")
    wid = sid * NC + lax.axis_index("c")
    base = wid * PER_TILE
    idx = (idx0, idx1)
    vals = (vals0, vals1)
    isem = (isem0, isem1)
    ssem = (ssem0, ssem1)

    stage = sid * STAGE
    for h in range(STAGE_NHOP):
        off = stage + h * STAGE_HOP
        pltpu.sync_copy(table_hbm.at[pl.ds(off, STAGE_HOP)], bounce)
        pltpu.sync_copy(bounce, table_sh.at[pl.ds(off, STAGE_HOP)])

    @pl.when(sid == 0)
    def _stage_tail():
        pltpu.sync_copy(table_hbm.at[pl.ds(NS * STAGE, STAGE_TAIL)],
                        bounce.at[pl.ds(0, STAGE_TAIL)])
        pltpu.sync_copy(bounce.at[pl.ds(0, STAGE_TAIL)],
                        table_sh.at[pl.ds(NS * STAGE, STAGE_TAIL)])

    plsc.subcore_barrier()

    for b in range(2):
        pltpu.async_copy(
            ids_hbm.at[pl.ds(base + b * CHUNK, CHUNK)], idx[b], isem[b])

    for i in range(NCHUNK):
        b = i % 2
        pltpu.make_async_copy(
            ids_hbm.at[pl.ds(base + i * CHUNK, CHUNK)], idx[b],
            isem[b]).wait()
        if i >= 2:
            pltpu.make_async_copy(
                vals[b], out_hbm.at[pl.ds(base + (i - 2) * CHUNK, CHUNK)],
                ssem[b]).wait()
        pltpu.async_copy(table_sh.at[idx[b]], vals[b], gsem).wait()
        pltpu.async_copy(
            vals[b], out_hbm.at[pl.ds(base + i * CHUNK, CHUNK)], ssem[b])
        if i + 2 < NCHUNK:
            pltpu.async_copy(
                ids_hbm.at[pl.ds(base + (i + 2) * CHUNK, CHUNK)], idx[b],
                isem[b])

    for i in range(NCHUNK - 2, NCHUNK):
        b = i % 2
        pltpu.make_async_copy(
            vals[b], out_hbm.at[pl.ds(base + i * CHUNK, CHUNK)],
            ssem[b]).wait()


def kernel(input_ids, mapping_table):
    flat_ids = input_ids.reshape(TOTAL)
    out = _gather_kernel(flat_ids, mapping_table)
    return out.reshape(BATCH, HIST)
